# Initial kernel scaffold; baseline (speedup 1.0000x reference)
import jax
import jax.numpy as jnp
from jax.experimental import pallas as pl


def kernel(atom_feat, atom_bond_feat, frag_feat, fbond_feat, atom_edge_index, atom_graph_ids, frag_edge_index, frag_graph_ids, eps, params):
    raise NotImplementedError("write your pallas kernel here")


# speedup vs baseline: 1.2796x; 1.2796x over previous
import functools

import jax
import jax.numpy as jnp
from jax import lax
from jax.experimental import pallas as pl
from jax.experimental.pallas import tpu as pltpu
from jax.experimental.pallas import tpu_sc as plsc

F32 = jnp.float32
_NC = 2
_NS = 16
_NW = _NC * _NS
_CH = 128

_SC_PARAMS = pltpu.CompilerParams(use_tc_tiling_on_sc=False)


def _rnd(n, m):
    return ((n + m - 1) // m) * m


def _bm(m, cap=2048):
    b = cap
    while m % b:
        b //= 2
    return b


@functools.lru_cache(maxsize=None)
def _make_gather(npad, d, epad):
    b = epad // _NW
    nch = b // _CH
    mesh = plsc.VectorSubcoreMesh(core_axis_name="c", subcore_axis_name="s")

    @functools.partial(
        pl.kernel,
        out_type=jax.ShapeDtypeStruct((epad, d), F32),
        mesh=mesh,
        compiler_params=_SC_PARAMS,
        scratch_types=[
            pltpu.VMEM((b,), jnp.int32),
            pltpu.VMEM((b, d), F32),
            pltpu.SemaphoreType.DMA,
        ],
    )
    def gather_k(table_hbm, idx_hbm, out_hbm, idx_v, rows_v, sem):
        wid = lax.axis_index("s") * _NC + lax.axis_index("c")
        pltpu.sync_copy(idx_hbm.at[pl.ds(wid * b, b)], idx_v)
        cps = []
        for j in range(nch):
            cps.append(pltpu.async_copy(
                table_hbm.at[idx_v.at[pl.ds(j * _CH, _CH)]],
                rows_v.at[pl.ds(j * _CH, _CH)], sem))
        for cp in cps:
            cp.wait()
        pltpu.sync_copy(rows_v, out_hbm.at[pl.ds(wid * b, b)])

    return gather_k


@functools.lru_cache(maxsize=None)
def _make_scatter(npad, d, epad):
    b = epad // _NW
    nch = b // _CH
    zr = npad // _NS
    mesh = plsc.VectorSubcoreMesh(core_axis_name="c", subcore_axis_name="s")

    @functools.partial(
        pl.kernel,
        out_type=jax.ShapeDtypeStruct((_NC * npad, d), F32),
        mesh=mesh,
        compiler_params=_SC_PARAMS,
        scratch_types=[
            pltpu.VMEM((b, d), F32),
            pltpu.VMEM((nch, _CH), jnp.int32),
            pltpu.VMEM_SHARED((npad, d), F32),
            pltpu.SemaphoreType.DMA,
        ],
    )
    def scatter_k(vals_hbm, idx_hbm, zeros_hbm, out_hbm, vals_v, idx_v, acc_sh, sem):
        c = lax.axis_index("c")
        s = lax.axis_index("s")
        wid = s * _NC + c
        pltpu.sync_copy(zeros_hbm.at[pl.ds(s * zr, zr)],
                        acc_sh.at[pl.ds(s * zr, zr)])
        plsc.subcore_barrier()
        pltpu.sync_copy(vals_hbm.at[pl.ds(wid * b, b)], vals_v)
        pltpu.sync_copy(idx_hbm.at[wid], idx_v)
        for j in range(nch):
            pltpu.sync_copy(vals_v.at[pl.ds(j * _CH, _CH)],
                            acc_sh.at[idx_v.at[j]], add=True)
        plsc.subcore_barrier()
        pltpu.sync_copy(acc_sh.at[pl.ds(s * zr, zr)],
                        out_hbm.at[pl.ds(c * npad + s * zr, zr)])

    return scatter_k


def _mm(x, wt, bias, act=None):
    m, k = x.shape
    n = wt.shape[1]
    bm = _bm(m)

    def body(x_ref, w_ref, b_ref, o_ref):
        y = jnp.dot(x_ref[...], w_ref[...], preferred_element_type=F32) + b_ref[...]
        if act == 'relu':
            y = jnp.maximum(y, 0.0)
        o_ref[...] = y

    return pl.pallas_call(
        body,
        grid=(m // bm,),
        in_specs=[pl.BlockSpec((bm, k), lambda i: (i, 0)),
                  pl.BlockSpec((k, n), lambda i: (0, 0)),
                  pl.BlockSpec((1, n), lambda i: (0, 0))],
        out_specs=pl.BlockSpec((bm, n), lambda i: (i, 0)),
        out_shape=jax.ShapeDtypeStruct((m, n), F32),
    )(x, wt, bias)


def _msg(hsrc, g, a_mat, h, kk):
    e = hsrc.shape[0]
    n = a_mat.shape[1]
    bm = _bm(e)

    def body(h_ref, g_ref, a_ref, o_ref):
        p = jnp.dot(h_ref[...], a_ref[...], preferred_element_type=F32)
        gg = g_ref[...]
        acc = p[:, kk * h:(kk + 1) * h]
        for k in range(kk):
            acc = acc + p[:, k * h:(k + 1) * h] * gg[:, k:k + 1]
        o_ref[...] = acc

    return pl.pallas_call(
        body,
        grid=(e // bm,),
        in_specs=[pl.BlockSpec((bm, h), lambda i: (i, 0)),
                  pl.BlockSpec((bm, g.shape[1]), lambda i: (i, 0)),
                  pl.BlockSpec((h, n), lambda i: (0, 0))],
        out_specs=pl.BlockSpec((bm, h), lambda i: (i, 0)),
        out_shape=jax.ShapeDtypeStruct((e, h), F32),
    )(hsrc, g, a_mat)


def _gru(acc, cnt, hprev, wit, wht, bi, bh):
    npad, d = hprev.shape
    bm = _bm(npad)
    nb = npad // bm

    def body(a0_ref, a1_ref, c0_ref, c1_ref, h_ref, wi_ref, wh_ref,
             bi_ref, bh_ref, o_ref):
        s = a0_ref[...] + a1_ref[...]
        c = jnp.maximum(c0_ref[:, :1] + c1_ref[:, :1], 1.0)
        m = jnp.maximum(s / c, 0.0)
        hh = h_ref[...]
        gi = jnp.dot(m, wi_ref[...], preferred_element_type=F32) + bi_ref[...]
        gh = jnp.dot(hh, wh_ref[...], preferred_element_type=F32) + bh_ref[...]
        r = jax.nn.sigmoid(gi[:, :d] + gh[:, :d])
        z = jax.nn.sigmoid(gi[:, d:2 * d] + gh[:, d:2 * d])
        nn = jnp.tanh(gi[:, 2 * d:] + r * gh[:, 2 * d:])
        o_ref[...] = (1.0 - z) * nn + z * hh

    return pl.pallas_call(
        body,
        grid=(nb,),
        in_specs=[pl.BlockSpec((bm, d), lambda i: (i, 0)),
                  pl.BlockSpec((bm, d), lambda i, nb=nb: (i + nb, 0)),
                  pl.BlockSpec((bm, 16), lambda i: (i, 0)),
                  pl.BlockSpec((bm, 16), lambda i, nb=nb: (i + nb, 0)),
                  pl.BlockSpec((bm, d), lambda i: (i, 0)),
                  pl.BlockSpec((d, 3 * d), lambda i: (0, 0)),
                  pl.BlockSpec((d, 3 * d), lambda i: (0, 0)),
                  pl.BlockSpec((1, 3 * d), lambda i: (0, 0)),
                  pl.BlockSpec((1, 3 * d), lambda i: (0, 0))],
        out_specs=pl.BlockSpec((bm, d), lambda i: (i, 0)),
        out_shape=jax.ShapeDtypeStruct((npad, d), F32),
    )(acc, acc, cnt, cnt, hprev, wit, wht, bi, bh)


def _frag_assemble(ff, wt, bias, acc, cnt):
    npad = ff.shape[0]
    k = ff.shape[1]
    d = wt.shape[1]
    bm = _bm(npad)
    nb = npad // bm

    def body(f_ref, w_ref, b_ref, a0_ref, a1_ref, c0_ref, c1_ref, o_ref):
        emb = jnp.dot(f_ref[...], w_ref[...], preferred_element_type=F32) + b_ref[...]
        s = a0_ref[...] + a1_ref[...]
        c = jnp.maximum(c0_ref[:, :1] + c1_ref[:, :1], 1.0)
        o_ref[...] = jnp.concatenate([emb, s / c], axis=1)

    return pl.pallas_call(
        body,
        grid=(nb,),
        in_specs=[pl.BlockSpec((bm, k), lambda i: (i, 0)),
                  pl.BlockSpec((k, d), lambda i: (0, 0)),
                  pl.BlockSpec((1, d), lambda i: (0, 0)),
                  pl.BlockSpec((bm, d), lambda i: (i, 0)),
                  pl.BlockSpec((bm, d), lambda i, nb=nb: (i + nb, 0)),
                  pl.BlockSpec((bm, 16), lambda i: (i, 0)),
                  pl.BlockSpec((bm, 16), lambda i, nb=nb: (i + nb, 0))],
        out_specs=pl.BlockSpec((bm, 2 * d), lambda i: (i, 0)),
        out_shape=jax.ShapeDtypeStruct((npad, 2 * d), F32),
    )(ff, wt, bias, acc, acc, cnt, cnt)


def _final(acc, cnt, wt, bias, eps, nb_real, latent):
    npad = acc.shape[0] // 2
    d = acc.shape[1]

    def body(a0_ref, a1_ref, c0_ref, c1_ref, w_ref, b_ref, e_ref,
             z_ref, mu_ref, lv_ref):
        s = a0_ref[...] + a1_ref[...]
        c = jnp.maximum(c0_ref[:, :1] + c1_ref[:, :1], 1.0)
        hm = (s / c)[:nb_real]
        x = jnp.dot(hm, w_ref[...], preferred_element_type=F32) + b_ref[...]
        mu = x[:, :latent]
        lv = x[:, latent:]
        std = jnp.exp(0.5 * lv)
        z_ref[...] = mu + e_ref[...] * std
        mu_ref[...] = mu
        lv_ref[...] = lv

    out = jax.ShapeDtypeStruct((nb_real, latent), F32)
    return pl.pallas_call(
        body,
        grid=(1,),
        in_specs=[pl.BlockSpec((npad, d), lambda i: (0, 0)),
                  pl.BlockSpec((npad, d), lambda i: (1, 0)),
                  pl.BlockSpec((npad, 16), lambda i: (0, 0)),
                  pl.BlockSpec((npad, 16), lambda i: (1, 0)),
                  pl.BlockSpec((d, 2 * latent), lambda i: (0, 0)),
                  pl.BlockSpec((1, 2 * latent), lambda i: (0, 0)),
                  pl.BlockSpec((nb_real, latent), lambda i: (0, 0))],
        out_specs=[pl.BlockSpec((nb_real, latent), lambda i: (0, 0))] * 3,
        out_shape=[out, out, out],
    )(acc, acc, cnt, cnt, wt, bias, eps)


def _edge_net_mat(e2w, e2b, h, k):
    a = e2w.reshape(h, h, k).transpose(0, 2, 1).reshape(h, k * h)
    return jnp.concatenate([a, e2b.reshape(h, h)], axis=1)


def _pad_idx(idx, epad, dump):
    return jnp.pad(idx, (0, epad - idx.shape[0]), constant_values=dump)


def _pad_idx3(idx, epad, dump):
    return jnp.pad(idx, (0, epad - idx.shape[0]),
                   constant_values=dump).reshape(_NW, -1, _CH)


def kernel(atom_feat, atom_bond_feat, frag_feat, fbond_feat, atom_edge_index,
           atom_graph_ids, frag_edge_index, frag_graph_ids, eps, params):
    p = params
    na, ea = atom_feat.shape[0], atom_edge_index.shape[1]
    nf, ef = frag_feat.shape[0], frag_edge_index.shape[1]
    nb = eps.shape[0]
    latent = eps.shape[1]
    ha = p['emb_atom_W'].shape[0]
    hf = 2 * p['emb_frag_W'].shape[0]
    ka = p['amp']['e1_W'].shape[0]
    kf = p['fmp']['e1_W'].shape[0]

    nap = _rnd(na + 1, 1024)
    nfp = _rnd(nf + 1, 1024)
    nbp = _rnd(nb + 1, 128)
    eap = _rnd(ea, _NW * _CH)
    efp = _rnd(ef, _NW * _CH)
    iap = _rnd(max(na, nap), _NW * _CH)
    ifp = _rnd(max(nf, nfp), _NW * _CH)

    a_src = _pad_idx(atom_edge_index[0], eap, nap - 1)
    a_dst = _pad_idx3(atom_edge_index[1], eap, nap - 1)
    f_src = _pad_idx(frag_edge_index[0], efp, nfp - 1)
    f_dst = _pad_idx3(frag_edge_index[1], efp, nfp - 1)
    a_gid = _pad_idx3(atom_graph_ids, iap, nfp - 1)
    f_gid = _pad_idx3(frag_graph_ids, ifp, nbp - 1)

    amp, fmp = p['amp'], p['fmp']
    w_bond = (amp['e1_W'] @ p['emb_bond_W']).T
    b_bond = (p['emb_bond_b'] @ amp['e1_W'].T + amp['e1_b'])[None]
    w_fbond = (fmp['e1_W'] @ p['emb_fbond_W']).T
    b_fbond = (p['emb_fbond_b'] @ fmp['e1_W'].T + fmp['e1_b'])[None]
    a_mat_a = _edge_net_mat(amp['e2_W'], amp['e2_b'], ha, ka)
    a_mat_f = _edge_net_mat(fmp['e2_W'], fmp['e2_b'], hf, kf)

    zeros_a = jnp.zeros((nap, ha), F32)
    zeros_f32 = jnp.zeros((nfp, ha), F32)
    zeros_f64 = jnp.zeros((nfp, hf), F32)
    zeros_b = jnp.zeros((nbp, hf), F32)
    zeros_ca = jnp.zeros((nap, 16), F32)
    zeros_cf = jnp.zeros((nfp, 16), F32)
    zeros_cb = jnp.zeros((nbp, 16), F32)

    cnt_a = _make_scatter(nap, 16, eap)(jnp.ones((eap, 16), F32), a_dst, zeros_ca)
    cnt_f = _make_scatter(nfp, 16, efp)(jnp.ones((efp, 16), F32), f_dst, zeros_cf)
    cnt_af = _make_scatter(nfp, 16, iap)(jnp.ones((iap, 16), F32), a_gid, zeros_cf)
    cnt_fb = _make_scatter(nbp, 16, ifp)(jnp.ones((ifp, 16), F32), f_gid, zeros_cb)

    af = jnp.pad(atom_feat, ((0, nap - na), (0, 0)))
    h = _mm(af, p['emb_atom_W'].T, p['emb_atom_b'][None])
    bf = jnp.pad(atom_bond_feat, ((0, eap - ea), (0, 0)))
    g_a = _mm(bf, w_bond, b_bond, act='relu')

    gather_a = _make_gather(nap, ha, eap)
    scat_a = _make_scatter(nap, ha, eap)
    wit_a, wht_a = amp['gru_Wih'].T, amp['gru_Whh'].T
    bi_a, bh_a = amp['gru_bih'][None], amp['gru_bhh'][None]
    for _ in range(2):
        hs = gather_a(h, a_src)
        msg = _msg(hs, g_a, a_mat_a, ha, ka)
        acc = scat_a(msg, a_dst, zeros_a)
        h = _gru(acc, cnt_a, h, wit_a, wht_a, bi_a, bh_a)

    h_pad = jnp.pad(h, ((0, iap - nap), (0, 0)))
    acc_af = _make_scatter(nfp, ha, iap)(h_pad, a_gid, zeros_f32)
    ffp = jnp.pad(frag_feat, ((0, nfp - nf), (0, 0)))
    hfr = _frag_assemble(ffp, p['emb_frag_W'].T, p['emb_frag_b'][None],
                         acc_af, cnt_af)

    fbf = jnp.pad(fbond_feat, ((0, efp - ef), (0, 0)))
    g_f = _mm(fbf, w_fbond, b_fbond, act='relu')
    gather_f = _make_gather(nfp, hf, efp)
    scat_f = _make_scatter(nfp, hf, efp)
    wit_f, wht_f = fmp['gru_Wih'].T, fmp['gru_Whh'].T
    bi_f, bh_f = fmp['gru_bih'][None], fmp['gru_bhh'][None]
    for _ in range(2):
        hs = gather_f(hfr, f_src)
        msg = _msg(hs, g_f, a_mat_f, hf, kf)
        acc = scat_f(msg, f_dst, zeros_f64)
        hfr = _gru(acc, cnt_f, hfr, wit_f, wht_f, bi_f, bh_f)

    hfr_pad = jnp.pad(hfr, ((0, ifp - nfp), (0, 0)))
    acc_fb = _make_scatter(nbp, hf, ifp)(hfr_pad, f_gid, zeros_b)
    z, mu, lv = _final(acc_fb, cnt_fb, p['enc_W'].T, p['enc_b'][None],
                       eps, nb, latent)
    return (z, mu, lv)


# speedup vs baseline: 1.2796x; 1.0000x over previous
import functools

import jax
import jax.numpy as jnp
from jax import lax
from jax.experimental import pallas as pl
from jax.experimental.pallas import tpu as pltpu
from jax.experimental.pallas import tpu_sc as plsc

F32 = jnp.float32
_NC = 2
_NS = 16
_NW = _NC * _NS
_CH = 128

_SC_PARAMS = pltpu.CompilerParams(use_tc_tiling_on_sc=False)


def _rnd(n, m):
    return ((n + m - 1) // m) * m


def _bm(m, cap=2048):
    b = cap
    while m % b:
        b //= 2
    return b


@functools.lru_cache(maxsize=None)
def _make_gather(npad, d, epad):
    b = epad // _NW
    nch = b // _CH
    mesh = plsc.VectorSubcoreMesh(core_axis_name="c", subcore_axis_name="s")

    @functools.partial(
        pl.kernel,
        out_type=jax.ShapeDtypeStruct((epad, d), F32),
        mesh=mesh,
        compiler_params=_SC_PARAMS,
        scratch_types=[
            pltpu.VMEM((b,), jnp.int32),
            pltpu.VMEM((b, d), F32),
            pltpu.SemaphoreType.DMA,
        ],
    )
    def gather_k(table_hbm, idx_hbm, out_hbm, idx_v, rows_v, sem):
        wid = lax.axis_index("s") * _NC + lax.axis_index("c")
        pltpu.sync_copy(idx_hbm.at[pl.ds(wid * b, b)], idx_v)
        cps = []
        for j in range(nch):
            cps.append(pltpu.async_copy(
                table_hbm.at[idx_v.at[pl.ds(j * _CH, _CH)]],
                rows_v.at[pl.ds(j * _CH, _CH)], sem))
        for cp in cps:
            cp.wait()
        pltpu.sync_copy(rows_v, out_hbm.at[pl.ds(wid * b, b)])

    return gather_k


@functools.lru_cache(maxsize=None)
def _make_scatter(npad, d, epad):
    b = epad // _NW
    nch = b // _CH
    zr = npad // _NS
    mesh = plsc.VectorSubcoreMesh(core_axis_name="c", subcore_axis_name="s")

    @functools.partial(
        pl.kernel,
        out_type=jax.ShapeDtypeStruct((_NC * npad, d), F32),
        mesh=mesh,
        compiler_params=_SC_PARAMS,
        scratch_types=[
            pltpu.VMEM((b, d), F32),
            pltpu.VMEM((nch, _CH), jnp.int32),
            pltpu.VMEM_SHARED((npad, d), F32),
            pltpu.SemaphoreType.DMA,
        ],
    )
    def scatter_k(vals_hbm, idx_hbm, zeros_hbm, out_hbm, vals_v, idx_v, acc_sh, sem):
        c = lax.axis_index("c")
        s = lax.axis_index("s")
        wid = s * _NC + c
        pltpu.sync_copy(zeros_hbm.at[pl.ds(s * zr, zr)],
                        acc_sh.at[pl.ds(s * zr, zr)])
        plsc.subcore_barrier()
        pltpu.sync_copy(vals_hbm.at[pl.ds(wid * b, b)], vals_v)
        pltpu.sync_copy(idx_hbm.at[wid], idx_v)
        for j in range(nch):
            pltpu.sync_copy(vals_v.at[pl.ds(j * _CH, _CH)],
                            acc_sh.at[idx_v.at[j]], add=True)
        plsc.subcore_barrier()
        pltpu.sync_copy(acc_sh.at[pl.ds(s * zr, zr)],
                        out_hbm.at[pl.ds(c * npad + s * zr, zr)])

    return scatter_k


def _mm(x, wt, bias, act=None):
    m, k = x.shape
    n = wt.shape[1]
    bm = _bm(m)

    def body(x_ref, w_ref, b_ref, o_ref):
        y = jnp.dot(x_ref[...], w_ref[...], preferred_element_type=F32) + b_ref[...]
        if act == 'relu':
            y = jnp.maximum(y, 0.0)
        o_ref[...] = y

    return pl.pallas_call(
        body,
        grid=(m // bm,),
        in_specs=[pl.BlockSpec((bm, k), lambda i: (i, 0)),
                  pl.BlockSpec((k, n), lambda i: (0, 0)),
                  pl.BlockSpec((1, n), lambda i: (0, 0))],
        out_specs=pl.BlockSpec((bm, n), lambda i: (i, 0)),
        out_shape=jax.ShapeDtypeStruct((m, n), F32),
    )(x, wt, bias)


def _msg(hsrc, g, a_mat, h, kk):
    e = hsrc.shape[0]
    n = a_mat.shape[1]
    bm = _bm(e)

    def body(h_ref, g_ref, a_ref, o_ref):
        p = jnp.dot(h_ref[...].astype(jnp.bfloat16), a_ref[...].astype(jnp.bfloat16),
                    preferred_element_type=F32)
        gg = g_ref[...]
        acc = p[:, kk * h:(kk + 1) * h]
        for k in range(kk):
            acc = acc + p[:, k * h:(k + 1) * h] * gg[:, k:k + 1]
        o_ref[...] = acc

    return pl.pallas_call(
        body,
        grid=(e // bm,),
        in_specs=[pl.BlockSpec((bm, h), lambda i: (i, 0)),
                  pl.BlockSpec((bm, g.shape[1]), lambda i: (i, 0)),
                  pl.BlockSpec((h, n), lambda i: (0, 0))],
        out_specs=pl.BlockSpec((bm, h), lambda i: (i, 0)),
        out_shape=jax.ShapeDtypeStruct((e, h), F32),
    )(hsrc, g, a_mat)


def _gru(acc, cnt, hprev, wit, wht, bi, bh):
    npad, d = hprev.shape
    bm = _bm(npad)
    nb = npad // bm

    def body(a0_ref, a1_ref, c0_ref, c1_ref, h_ref, wi_ref, wh_ref,
             bi_ref, bh_ref, o_ref):
        s = a0_ref[...] + a1_ref[...]
        c = jnp.maximum(c0_ref[:, :1] + c1_ref[:, :1], 1.0)
        m = jnp.maximum(s / c, 0.0)
        hh = h_ref[...]
        gi = jnp.dot(m, wi_ref[...], preferred_element_type=F32) + bi_ref[...]
        gh = jnp.dot(hh, wh_ref[...], preferred_element_type=F32) + bh_ref[...]
        r = jax.nn.sigmoid(gi[:, :d] + gh[:, :d])
        z = jax.nn.sigmoid(gi[:, d:2 * d] + gh[:, d:2 * d])
        nn = jnp.tanh(gi[:, 2 * d:] + r * gh[:, 2 * d:])
        o_ref[...] = (1.0 - z) * nn + z * hh

    return pl.pallas_call(
        body,
        grid=(nb,),
        in_specs=[pl.BlockSpec((bm, d), lambda i: (i, 0)),
                  pl.BlockSpec((bm, d), lambda i, nb=nb: (i + nb, 0)),
                  pl.BlockSpec((bm, 16), lambda i: (i, 0)),
                  pl.BlockSpec((bm, 16), lambda i, nb=nb: (i + nb, 0)),
                  pl.BlockSpec((bm, d), lambda i: (i, 0)),
                  pl.BlockSpec((d, 3 * d), lambda i: (0, 0)),
                  pl.BlockSpec((d, 3 * d), lambda i: (0, 0)),
                  pl.BlockSpec((1, 3 * d), lambda i: (0, 0)),
                  pl.BlockSpec((1, 3 * d), lambda i: (0, 0))],
        out_specs=pl.BlockSpec((bm, d), lambda i: (i, 0)),
        out_shape=jax.ShapeDtypeStruct((npad, d), F32),
    )(acc, acc, cnt, cnt, hprev, wit, wht, bi, bh)


def _frag_assemble(ff, wt, bias, acc, cnt):
    npad = ff.shape[0]
    k = ff.shape[1]
    d = wt.shape[1]
    bm = _bm(npad)
    nb = npad // bm

    def body(f_ref, w_ref, b_ref, a0_ref, a1_ref, c0_ref, c1_ref, o_ref):
        emb = jnp.dot(f_ref[...], w_ref[...], preferred_element_type=F32) + b_ref[...]
        s = a0_ref[...] + a1_ref[...]
        c = jnp.maximum(c0_ref[:, :1] + c1_ref[:, :1], 1.0)
        o_ref[...] = jnp.concatenate([emb, s / c], axis=1)

    return pl.pallas_call(
        body,
        grid=(nb,),
        in_specs=[pl.BlockSpec((bm, k), lambda i: (i, 0)),
                  pl.BlockSpec((k, d), lambda i: (0, 0)),
                  pl.BlockSpec((1, d), lambda i: (0, 0)),
                  pl.BlockSpec((bm, d), lambda i: (i, 0)),
                  pl.BlockSpec((bm, d), lambda i, nb=nb: (i + nb, 0)),
                  pl.BlockSpec((bm, 16), lambda i: (i, 0)),
                  pl.BlockSpec((bm, 16), lambda i, nb=nb: (i + nb, 0))],
        out_specs=pl.BlockSpec((bm, 2 * d), lambda i: (i, 0)),
        out_shape=jax.ShapeDtypeStruct((npad, 2 * d), F32),
    )(ff, wt, bias, acc, acc, cnt, cnt)


def _final(acc, cnt, wt, bias, eps, nb_real, latent):
    npad = acc.shape[0] // 2
    d = acc.shape[1]

    def body(a0_ref, a1_ref, c0_ref, c1_ref, w_ref, b_ref, e_ref,
             z_ref, mu_ref, lv_ref):
        s = a0_ref[...] + a1_ref[...]
        c = jnp.maximum(c0_ref[:, :1] + c1_ref[:, :1], 1.0)
        hm = (s / c)[:nb_real]
        x = jnp.dot(hm, w_ref[...], preferred_element_type=F32) + b_ref[...]
        mu = x[:, :latent]
        lv = x[:, latent:]
        std = jnp.exp(0.5 * lv)
        z_ref[...] = mu + e_ref[...] * std
        mu_ref[...] = mu
        lv_ref[...] = lv

    out = jax.ShapeDtypeStruct((nb_real, latent), F32)
    return pl.pallas_call(
        body,
        grid=(1,),
        in_specs=[pl.BlockSpec((npad, d), lambda i: (0, 0)),
                  pl.BlockSpec((npad, d), lambda i: (1, 0)),
                  pl.BlockSpec((npad, 16), lambda i: (0, 0)),
                  pl.BlockSpec((npad, 16), lambda i: (1, 0)),
                  pl.BlockSpec((d, 2 * latent), lambda i: (0, 0)),
                  pl.BlockSpec((1, 2 * latent), lambda i: (0, 0)),
                  pl.BlockSpec((nb_real, latent), lambda i: (0, 0))],
        out_specs=[pl.BlockSpec((nb_real, latent), lambda i: (0, 0))] * 3,
        out_shape=[out, out, out],
    )(acc, acc, cnt, cnt, wt, bias, eps)


def _edge_net_mat(e2w, e2b, h, k):
    a = e2w.reshape(h, h, k).transpose(0, 2, 1).reshape(h, k * h)
    return jnp.concatenate([a, e2b.reshape(h, h)], axis=1)


def _pad_idx(idx, epad, dump):
    return jnp.pad(idx, (0, epad - idx.shape[0]), constant_values=dump)


def _pad_idx3(idx, epad, dump):
    return jnp.pad(idx, (0, epad - idx.shape[0]),
                   constant_values=dump).reshape(_NW, -1, _CH)


def kernel(atom_feat, atom_bond_feat, frag_feat, fbond_feat, atom_edge_index,
           atom_graph_ids, frag_edge_index, frag_graph_ids, eps, params):
    p = params
    na, ea = atom_feat.shape[0], atom_edge_index.shape[1]
    nf, ef = frag_feat.shape[0], frag_edge_index.shape[1]
    nb = eps.shape[0]
    latent = eps.shape[1]
    ha = p['emb_atom_W'].shape[0]
    hf = 2 * p['emb_frag_W'].shape[0]
    ka = p['amp']['e1_W'].shape[0]
    kf = p['fmp']['e1_W'].shape[0]

    nap = _rnd(na + 1, 1024)
    nfp = _rnd(nf + 1, 1024)
    nbp = _rnd(nb + 1, 128)
    eap = _rnd(ea, _NW * _CH)
    efp = _rnd(ef, _NW * _CH)
    iap = _rnd(max(na, nap), _NW * _CH)
    ifp = _rnd(max(nf, nfp), _NW * _CH)

    a_src = _pad_idx(atom_edge_index[0], eap, nap - 1)
    a_dst = _pad_idx3(atom_edge_index[1], eap, nap - 1)
    f_src = _pad_idx(frag_edge_index[0], efp, nfp - 1)
    f_dst = _pad_idx3(frag_edge_index[1], efp, nfp - 1)
    a_gid = _pad_idx3(atom_graph_ids, iap, nfp - 1)
    f_gid = _pad_idx3(frag_graph_ids, ifp, nbp - 1)

    amp, fmp = p['amp'], p['fmp']
    w_bond = (amp['e1_W'] @ p['emb_bond_W']).T
    b_bond = (p['emb_bond_b'] @ amp['e1_W'].T + amp['e1_b'])[None]
    w_fbond = (fmp['e1_W'] @ p['emb_fbond_W']).T
    b_fbond = (p['emb_fbond_b'] @ fmp['e1_W'].T + fmp['e1_b'])[None]
    a_mat_a = _edge_net_mat(amp['e2_W'], amp['e2_b'], ha, ka)
    a_mat_f = _edge_net_mat(fmp['e2_W'], fmp['e2_b'], hf, kf)

    zeros_a = jnp.zeros((nap, ha), F32)
    zeros_f32 = jnp.zeros((nfp, ha), F32)
    zeros_f64 = jnp.zeros((nfp, hf), F32)
    zeros_b = jnp.zeros((nbp, hf), F32)
    zeros_ca = jnp.zeros((nap, 16), F32)
    zeros_cf = jnp.zeros((nfp, 16), F32)
    zeros_cb = jnp.zeros((nbp, 16), F32)

    cnt_a = _make_scatter(nap, 16, eap)(jnp.ones((eap, 16), F32), a_dst, zeros_ca)
    cnt_f = _make_scatter(nfp, 16, efp)(jnp.ones((efp, 16), F32), f_dst, zeros_cf)
    cnt_af = _make_scatter(nfp, 16, iap)(jnp.ones((iap, 16), F32), a_gid, zeros_cf)
    cnt_fb = _make_scatter(nbp, 16, ifp)(jnp.ones((ifp, 16), F32), f_gid, zeros_cb)

    af = jnp.pad(atom_feat, ((0, nap - na), (0, 0)))
    h = _mm(af, p['emb_atom_W'].T, p['emb_atom_b'][None])
    bf = jnp.pad(atom_bond_feat, ((0, eap - ea), (0, 0)))
    g_a = _mm(bf, w_bond, b_bond, act='relu')

    gather_a = _make_gather(nap, ha, eap)
    scat_a = _make_scatter(nap, ha, eap)
    wit_a, wht_a = amp['gru_Wih'].T, amp['gru_Whh'].T
    bi_a, bh_a = amp['gru_bih'][None], amp['gru_bhh'][None]
    for _ in range(2):
        hs = gather_a(h, a_src)
        msg = _msg(hs, g_a, a_mat_a, ha, ka)
        acc = scat_a(msg, a_dst, zeros_a)
        h = _gru(acc, cnt_a, h, wit_a, wht_a, bi_a, bh_a)

    h_pad = jnp.pad(h, ((0, iap - nap), (0, 0)))
    acc_af = _make_scatter(nfp, ha, iap)(h_pad, a_gid, zeros_f32)
    ffp = jnp.pad(frag_feat, ((0, nfp - nf), (0, 0)))
    hfr = _frag_assemble(ffp, p['emb_frag_W'].T, p['emb_frag_b'][None],
                         acc_af, cnt_af)

    fbf = jnp.pad(fbond_feat, ((0, efp - ef), (0, 0)))
    g_f = _mm(fbf, w_fbond, b_fbond, act='relu')
    gather_f = _make_gather(nfp, hf, efp)
    scat_f = _make_scatter(nfp, hf, efp)
    wit_f, wht_f = fmp['gru_Wih'].T, fmp['gru_Whh'].T
    bi_f, bh_f = fmp['gru_bih'][None], fmp['gru_bhh'][None]
    for _ in range(2):
        hs = gather_f(hfr, f_src)
        msg = _msg(hs, g_f, a_mat_f, hf, kf)
        acc = scat_f(msg, f_dst, zeros_f64)
        hfr = _gru(acc, cnt_f, hfr, wit_f, wht_f, bi_f, bh_f)

    hfr_pad = jnp.pad(hfr, ((0, ifp - nfp), (0, 0)))
    acc_fb = _make_scatter(nbp, hf, ifp)(hfr_pad, f_gid, zeros_b)
    z, mu, lv = _final(acc_fb, cnt_fb, p['enc_W'].T, p['enc_b'][None],
                       eps, nb, latent)
    return (z, mu, lv)


# speedup vs baseline: 2.2408x; 1.7512x over previous
import functools

import jax
import jax.numpy as jnp
from jax import lax
from jax.experimental import pallas as pl
from jax.experimental.pallas import tpu as pltpu
from jax.experimental.pallas import tpu_sc as plsc

F32 = jnp.float32
_NC = 2
_NS = 16
_NW = _NC * _NS
_CH = 128

_SC_PARAMS = pltpu.CompilerParams(use_tc_tiling_on_sc=False)


def _rnd(n, m):
    return ((n + m - 1) // m) * m


def _bm(m, cap=2048):
    b = cap
    while m % b:
        b //= 2
    return b


@functools.lru_cache(maxsize=None)
def _make_gather(npad, d, epad):
    b = epad // _NW
    nch = b // _CH
    mesh = plsc.VectorSubcoreMesh(core_axis_name="c", subcore_axis_name="s")

    @functools.partial(
        pl.kernel,
        out_type=jax.ShapeDtypeStruct((epad, d), F32),
        mesh=mesh,
        compiler_params=_SC_PARAMS,
        scratch_types=[
            pltpu.VMEM((b,), jnp.int32),
            pltpu.VMEM((b, d), F32),
            pltpu.SemaphoreType.DMA,
        ],
    )
    def gather_k(table_hbm, idx_hbm, out_hbm, idx_v, rows_v, sem):
        wid = lax.axis_index("s") * _NC + lax.axis_index("c")
        pltpu.sync_copy(idx_hbm.at[pl.ds(wid * b, b)], idx_v)
        cps = []
        for j in range(nch):
            cps.append(pltpu.async_copy(
                table_hbm.at[idx_v.at[pl.ds(j * _CH, _CH)]],
                rows_v.at[pl.ds(j * _CH, _CH)], sem))
        for cp in cps:
            cp.wait()
        pltpu.sync_copy(rows_v, out_hbm.at[pl.ds(wid * b, b)])

    return gather_k


@functools.lru_cache(maxsize=None)
def _make_scatter(npad, d, epad):
    b = epad // _NW
    nch = b // _CH
    zr = npad // _NS
    mesh = plsc.VectorSubcoreMesh(core_axis_name="c", subcore_axis_name="s")

    @functools.partial(
        pl.kernel,
        out_type=jax.ShapeDtypeStruct((_NC * npad, d), F32),
        mesh=mesh,
        compiler_params=_SC_PARAMS,
        scratch_types=[
            pltpu.VMEM((b, d), F32),
            pltpu.VMEM((nch, _CH), jnp.int32),
            pltpu.VMEM_SHARED((npad, d), F32),
            pltpu.SemaphoreType.DMA,
        ],
    )
    def scatter_k(vals_hbm, idx_hbm, zeros_hbm, out_hbm, vals_v, idx_v, acc_sh, sem):
        c = lax.axis_index("c")
        s = lax.axis_index("s")
        wid = s * _NC + c
        pltpu.sync_copy(zeros_hbm.at[pl.ds(s * zr, zr)],
                        acc_sh.at[pl.ds(s * zr, zr)])
        plsc.subcore_barrier()
        pltpu.sync_copy(vals_hbm.at[pl.ds(wid * b, b)], vals_v)
        pltpu.sync_copy(idx_hbm.at[wid], idx_v)
        for j in range(nch):
            pltpu.sync_copy(vals_v.at[pl.ds(j * _CH, _CH)],
                            acc_sh.at[idx_v.at[j]], add=True)
        plsc.subcore_barrier()
        pltpu.sync_copy(acc_sh.at[pl.ds(s * zr, zr)],
                        out_hbm.at[pl.ds(c * npad + s * zr, zr)])

    return scatter_k


def _mm(x, wt, bias, act=None):
    m, k = x.shape
    n = wt.shape[1]
    bm = _bm(m)

    def body(x_ref, w_ref, b_ref, o_ref):
        y = jnp.dot(x_ref[...], w_ref[...], preferred_element_type=F32) + b_ref[...]
        if act == 'relu':
            y = jnp.maximum(y, 0.0)
        o_ref[...] = y

    return pl.pallas_call(
        body,
        grid=(m // bm,),
        in_specs=[pl.BlockSpec((bm, k), lambda i: (i, 0)),
                  pl.BlockSpec((k, n), lambda i: (0, 0)),
                  pl.BlockSpec((1, n), lambda i: (0, 0))],
        out_specs=pl.BlockSpec((bm, n), lambda i: (i, 0)),
        out_shape=jax.ShapeDtypeStruct((m, n), F32),
    )(x, wt, bias)


def _msg(hsrc, g1, a_mat, r_mat, s_mat, h):
    e = hsrc.shape[0]
    n = a_mat.shape[1]
    bm = _bm(e)
    bf16 = jnp.bfloat16

    def body(h_ref, g_ref, a_ref, r_ref, s_ref, o_ref):
        p = jnp.dot(h_ref[...].astype(bf16), a_ref[...],
                    preferred_element_type=F32)
        t = jnp.dot(g_ref[...].astype(bf16), r_ref[...],
                    preferred_element_type=F32)
        q = (p * t).astype(bf16)
        o_ref[...] = jnp.dot(q, s_ref[...], preferred_element_type=F32)

    return pl.pallas_call(
        body,
        grid=(e // bm,),
        in_specs=[pl.BlockSpec((bm, hsrc.shape[1]), lambda i: (i, 0)),
                  pl.BlockSpec((bm, g1.shape[1]), lambda i: (i, 0)),
                  pl.BlockSpec((a_mat.shape[0], n), lambda i: (0, 0)),
                  pl.BlockSpec((r_mat.shape[0], n), lambda i: (0, 0)),
                  pl.BlockSpec((n, h), lambda i: (0, 0))],
        out_specs=pl.BlockSpec((bm, h), lambda i: (i, 0)),
        out_shape=jax.ShapeDtypeStruct((e, h), F32),
    )(hsrc, g1, a_mat, r_mat, s_mat)


def _gru(acc, cnt, hprev, wit, wht, bi, bh):
    npad, d = hprev.shape
    bm = _bm(npad)
    nb = npad // bm

    def body(a0_ref, a1_ref, c0_ref, c1_ref, h_ref, wi_ref, wh_ref,
             bi_ref, bh_ref, o_ref):
        s = a0_ref[...] + a1_ref[...]
        c = jnp.maximum(c0_ref[:, :1] + c1_ref[:, :1], 1.0)
        m = jnp.maximum(s / c, 0.0)
        hh = h_ref[...]
        gi = jnp.dot(m, wi_ref[...], preferred_element_type=F32) + bi_ref[...]
        gh = jnp.dot(hh, wh_ref[...], preferred_element_type=F32) + bh_ref[...]
        r = jax.nn.sigmoid(gi[:, :d] + gh[:, :d])
        z = jax.nn.sigmoid(gi[:, d:2 * d] + gh[:, d:2 * d])
        nn = jnp.tanh(gi[:, 2 * d:] + r * gh[:, 2 * d:])
        o_ref[...] = (1.0 - z) * nn + z * hh

    return pl.pallas_call(
        body,
        grid=(nb,),
        in_specs=[pl.BlockSpec((bm, d), lambda i: (i, 0)),
                  pl.BlockSpec((bm, d), lambda i, nb=nb: (i + nb, 0)),
                  pl.BlockSpec((bm, 16), lambda i: (i, 0)),
                  pl.BlockSpec((bm, 16), lambda i, nb=nb: (i + nb, 0)),
                  pl.BlockSpec((bm, d), lambda i: (i, 0)),
                  pl.BlockSpec((d, 3 * d), lambda i: (0, 0)),
                  pl.BlockSpec((d, 3 * d), lambda i: (0, 0)),
                  pl.BlockSpec((1, 3 * d), lambda i: (0, 0)),
                  pl.BlockSpec((1, 3 * d), lambda i: (0, 0))],
        out_specs=pl.BlockSpec((bm, d), lambda i: (i, 0)),
        out_shape=jax.ShapeDtypeStruct((npad, d), F32),
    )(acc, acc, cnt, cnt, hprev, wit, wht, bi, bh)


def _frag_assemble(ff, wt, bias, acc, cnt):
    npad = ff.shape[0]
    k = ff.shape[1]
    d = wt.shape[1]
    bm = _bm(npad)
    nb = npad // bm

    def body(f_ref, w_ref, b_ref, a0_ref, a1_ref, c0_ref, c1_ref, o_ref):
        emb = jnp.dot(f_ref[...], w_ref[...], preferred_element_type=F32) + b_ref[...]
        s = a0_ref[...] + a1_ref[...]
        c = jnp.maximum(c0_ref[:, :1] + c1_ref[:, :1], 1.0)
        o_ref[...] = jnp.concatenate([emb, s / c], axis=1)

    return pl.pallas_call(
        body,
        grid=(nb,),
        in_specs=[pl.BlockSpec((bm, k), lambda i: (i, 0)),
                  pl.BlockSpec((k, d), lambda i: (0, 0)),
                  pl.BlockSpec((1, d), lambda i: (0, 0)),
                  pl.BlockSpec((bm, d), lambda i: (i, 0)),
                  pl.BlockSpec((bm, d), lambda i, nb=nb: (i + nb, 0)),
                  pl.BlockSpec((bm, 16), lambda i: (i, 0)),
                  pl.BlockSpec((bm, 16), lambda i, nb=nb: (i + nb, 0))],
        out_specs=pl.BlockSpec((bm, 2 * d), lambda i: (i, 0)),
        out_shape=jax.ShapeDtypeStruct((npad, 2 * d), F32),
    )(ff, wt, bias, acc, acc, cnt, cnt)


def _final(acc, cnt, wt, bias, eps, nb_real, latent):
    npad = acc.shape[0] // 2
    d = acc.shape[1]

    def body(a0_ref, a1_ref, c0_ref, c1_ref, w_ref, b_ref, e_ref,
             z_ref, mu_ref, lv_ref):
        s = a0_ref[...] + a1_ref[...]
        c = jnp.maximum(c0_ref[:, :1] + c1_ref[:, :1], 1.0)
        hm = (s / c)[:nb_real]
        x = jnp.dot(hm, w_ref[...], preferred_element_type=F32) + b_ref[...]
        mu = x[:, :latent]
        lv = x[:, latent:]
        std = jnp.exp(0.5 * lv)
        z_ref[...] = mu + e_ref[...] * std
        mu_ref[...] = mu
        lv_ref[...] = lv

    out = jax.ShapeDtypeStruct((nb_real, latent), F32)
    return pl.pallas_call(
        body,
        grid=(1,),
        in_specs=[pl.BlockSpec((npad, d), lambda i: (0, 0)),
                  pl.BlockSpec((npad, d), lambda i: (1, 0)),
                  pl.BlockSpec((npad, 16), lambda i: (0, 0)),
                  pl.BlockSpec((npad, 16), lambda i: (1, 0)),
                  pl.BlockSpec((d, 2 * latent), lambda i: (0, 0)),
                  pl.BlockSpec((1, 2 * latent), lambda i: (0, 0)),
                  pl.BlockSpec((nb_real, latent), lambda i: (0, 0))],
        out_specs=[pl.BlockSpec((nb_real, latent), lambda i: (0, 0))] * 3,
        out_shape=[out, out, out],
    )(acc, acc, cnt, cnt, wt, bias, eps)


def _edge_net_mat(e2w, e2b, h, k):
    a = e2w.reshape(h, h, k).transpose(0, 2, 1).reshape(h, k * h)
    return jnp.concatenate([a, e2b.reshape(h, h)], axis=1)


def _pad_idx(idx, epad, dump):
    return jnp.pad(idx, (0, epad - idx.shape[0]), constant_values=dump)


def _pad_idx3(idx, epad, dump):
    return jnp.pad(idx, (0, epad - idx.shape[0]),
                   constant_values=dump).reshape(_NW, -1, _CH)


def kernel(atom_feat, atom_bond_feat, frag_feat, fbond_feat, atom_edge_index,
           atom_graph_ids, frag_edge_index, frag_graph_ids, eps, params):
    p = params
    na, ea = atom_feat.shape[0], atom_edge_index.shape[1]
    nf, ef = frag_feat.shape[0], frag_edge_index.shape[1]
    nb = eps.shape[0]
    latent = eps.shape[1]
    ha = p['emb_atom_W'].shape[0]
    hf = 2 * p['emb_frag_W'].shape[0]
    ka = p['amp']['e1_W'].shape[0]
    kf = p['fmp']['e1_W'].shape[0]

    nap = _rnd(na + 1, 1024)
    nfp = _rnd(nf + 1, 1024)
    nbp = _rnd(nb + 1, 128)
    eap = _rnd(ea, _NW * _CH)
    efp = _rnd(ef, _NW * _CH)
    iap = _rnd(max(na, nap), _NW * _CH)
    ifp = _rnd(max(nf, nfp), _NW * _CH)

    a_src = _pad_idx(atom_edge_index[0], eap, nap - 1)
    a_dst = _pad_idx3(atom_edge_index[1], eap, nap - 1)
    f_src = _pad_idx(frag_edge_index[0], efp, nfp - 1)
    f_dst = _pad_idx3(frag_edge_index[1], efp, nfp - 1)
    a_gid = _pad_idx3(atom_graph_ids, iap, nfp - 1)
    f_gid = _pad_idx3(frag_graph_ids, ifp, nbp - 1)

    amp, fmp = p['amp'], p['fmp']
    bf16 = jnp.bfloat16
    w_bond = (amp['e1_W'] @ p['emb_bond_W']).T
    b_bond = (p['emb_bond_b'] @ amp['e1_W'].T + amp['e1_b'])[None]
    w_fbond = (fmp['e1_W'] @ p['emb_fbond_W']).T
    b_fbond = (p['emb_fbond_b'] @ fmp['e1_W'].T + fmp['e1_b'])[None]
    w_bond = jnp.pad(w_bond, ((0, 0), (0, 1)))
    b_bond = jnp.concatenate([b_bond, jnp.ones((1, 1), F32)], axis=1)
    w_fbond = jnp.pad(w_fbond, ((0, 0), (0, 1)))
    b_fbond = jnp.concatenate([b_fbond, jnp.ones((1, 1), F32)], axis=1)
    a_mat_a = _edge_net_mat(amp['e2_W'], amp['e2_b'], ha, ka).astype(bf16)
    a_mat_f = _edge_net_mat(fmp['e2_W'], fmp['e2_b'], hf, kf).astype(bf16)
    r_a = jnp.kron(jnp.eye(ka + 1, dtype=F32), jnp.ones((1, ha), F32)).astype(bf16)
    s_a = jnp.tile(jnp.eye(ha, dtype=F32), (ka + 1, 1)).astype(bf16)
    r_f = jnp.kron(jnp.eye(kf + 1, dtype=F32), jnp.ones((1, hf), F32)).astype(bf16)
    s_f = jnp.tile(jnp.eye(hf, dtype=F32), (kf + 1, 1)).astype(bf16)

    zeros_a = jnp.zeros((nap, ha), F32)
    zeros_f32 = jnp.zeros((nfp, ha), F32)
    zeros_f64 = jnp.zeros((nfp, hf), F32)
    zeros_b = jnp.zeros((nbp, hf), F32)
    zeros_ca = jnp.zeros((nap, 16), F32)
    zeros_cf = jnp.zeros((nfp, 16), F32)
    zeros_cb = jnp.zeros((nbp, 16), F32)

    cnt_a = _make_scatter(nap, 16, eap)(jnp.ones((eap, 16), F32), a_dst, zeros_ca)
    cnt_f = _make_scatter(nfp, 16, efp)(jnp.ones((efp, 16), F32), f_dst, zeros_cf)
    cnt_af = _make_scatter(nfp, 16, iap)(jnp.ones((iap, 16), F32), a_gid, zeros_cf)
    cnt_fb = _make_scatter(nbp, 16, ifp)(jnp.ones((ifp, 16), F32), f_gid, zeros_cb)

    af = jnp.pad(atom_feat, ((0, nap - na), (0, 0)))
    h = _mm(af, p['emb_atom_W'].T, p['emb_atom_b'][None])
    bf = jnp.pad(atom_bond_feat, ((0, eap - ea), (0, 0)))
    g_a = _mm(bf, w_bond, b_bond, act='relu')

    gather_a = _make_gather(nap, ha, eap)
    scat_a = _make_scatter(nap, ha, eap)
    wit_a, wht_a = amp['gru_Wih'].T, amp['gru_Whh'].T
    bi_a, bh_a = amp['gru_bih'][None], amp['gru_bhh'][None]
    for _ in range(2):
        hs = gather_a(h, a_src)
        msg = _msg(hs, g_a, a_mat_a, r_a, s_a, ha)
        acc = scat_a(msg, a_dst, zeros_a)
        h = _gru(acc, cnt_a, h, wit_a, wht_a, bi_a, bh_a)

    h_pad = jnp.pad(h, ((0, iap - nap), (0, 0)))
    acc_af = _make_scatter(nfp, ha, iap)(h_pad, a_gid, zeros_f32)
    ffp = jnp.pad(frag_feat, ((0, nfp - nf), (0, 0)))
    hfr = _frag_assemble(ffp, p['emb_frag_W'].T, p['emb_frag_b'][None],
                         acc_af, cnt_af)

    fbf = jnp.pad(fbond_feat, ((0, efp - ef), (0, 0)))
    g_f = _mm(fbf, w_fbond, b_fbond, act='relu')
    gather_f = _make_gather(nfp, hf, efp)
    scat_f = _make_scatter(nfp, hf, efp)
    wit_f, wht_f = fmp['gru_Wih'].T, fmp['gru_Whh'].T
    bi_f, bh_f = fmp['gru_bih'][None], fmp['gru_bhh'][None]
    for _ in range(2):
        hs = gather_f(hfr, f_src)
        msg = _msg(hs, g_f, a_mat_f, r_f, s_f, hf)
        acc = scat_f(msg, f_dst, zeros_f64)
        hfr = _gru(acc, cnt_f, hfr, wit_f, wht_f, bi_f, bh_f)

    hfr_pad = jnp.pad(hfr, ((0, ifp - nfp), (0, 0)))
    acc_fb = _make_scatter(nbp, hf, ifp)(hfr_pad, f_gid, zeros_b)
    z, mu, lv = _final(acc_fb, cnt_fb, p['enc_W'].T, p['enc_b'][None],
                       eps, nb, latent)
    return (z, mu, lv)
